# Initial kernel scaffold; baseline (speedup 1.0000x reference)
import functools

import jax
import jax.numpy as jnp
from jax import lax
from jax.experimental import pallas as pl
from jax.experimental.pallas import tpu as pltpu
from jax.experimental.pallas import tpu_sc as plsc

B = 4096
N = 200
E = 128

_NUM_CORES = 2
_NUM_SUBCORES = 16
_LANES = 16
_NUM_WORKERS = _NUM_CORES * _NUM_SUBCORES
_B_PER_W = B // _NUM_WORKERS


@functools.partial(
    pl.kernel,
    mesh=plsc.VectorSubcoreMesh(core_axis_name="c", subcore_axis_name="s"),
    out_type=jax.ShapeDtypeStruct((B, E), jnp.float32),
    scratch_types=[
        pltpu.VMEM((_B_PER_W,), jnp.int32),
        pltpu.VMEM((_B_PER_W, E), jnp.float32),
        pltpu.SemaphoreType.DMA,
    ],
)
def _gather_rows(table_hbm, idx_hbm, out_hbm, idx_v, rows_v, sem):
    wid = lax.axis_index("s") * _NUM_CORES + lax.axis_index("c")
    base = wid * _B_PER_W
    pltpu.sync_copy(idx_hbm.at[pl.ds(base, _B_PER_W)], idx_v)
    lane = lax.iota(jnp.int32, (_LANES,), 0)
    for i in range(_B_PER_W // _LANES):
        sl = pl.ds(i * _LANES, _LANES)
        idx_v[sl] = idx_v[sl] + (base + i * _LANES) * N + lane * N
    pltpu.async_copy(table_hbm.at[idx_v], rows_v, sem).wait()
    pltpu.sync_copy(rows_v, out_hbm.at[pl.ds(base, _B_PER_W)])


def kernel(nodes_or_embeddings, current_node):
    cn = current_node
    if cn.ndim > 1:
        cn = jnp.squeeze(cn, axis=-1)
    cn = cn.astype(jnp.int32)
    table = nodes_or_embeddings.reshape(B * N, E)
    out = _gather_rows(table, cn)
    return out.reshape(B, 1, E)


# speedup vs baseline: 4.0213x; 4.0213x over previous
import functools

import jax
import jax.numpy as jnp
from jax import lax
from jax.experimental import pallas as pl
from jax.experimental.pallas import tpu as pltpu
from jax.experimental.pallas import tpu_sc as plsc

B = 4096
N = 200
E = 128

_NUM_CORES = 2
_NUM_SUBCORES = 16
_LANES = 16
_NUM_WORKERS = _NUM_CORES * _NUM_SUBCORES
_B_PER_W = B // _NUM_WORKERS


@functools.partial(
    pl.kernel,
    mesh=plsc.VectorSubcoreMesh(core_axis_name="c", subcore_axis_name="s"),
    out_type=jax.ShapeDtypeStruct((B, E), jnp.float32),
    scratch_types=[
        pltpu.VMEM((_B_PER_W,), jnp.int32),
        pltpu.VMEM((_B_PER_W, E), jnp.float32),
        pltpu.SemaphoreType.DMA,
    ],
)
def _gather_rows(table_hbm, idx_hbm, out_hbm, idx_v, rows_v, sem):
    wid = lax.axis_index("s") * _NUM_CORES + lax.axis_index("c")
    base = wid * _B_PER_W
    pltpu.sync_copy(idx_hbm.at[pl.ds(base, _B_PER_W)], idx_v)
    lane = lax.broadcasted_iota(jnp.int32, (_LANES,), 0)
    for i in range(_B_PER_W // _LANES):
        sl = pl.ds(i * _LANES, _LANES)
        idx_v[sl] = idx_v[sl] + (base + i * _LANES) * N + lane * N
    pltpu.async_copy(table_hbm.at[idx_v], rows_v, sem).wait()
    pltpu.sync_copy(rows_v, out_hbm.at[pl.ds(base, _B_PER_W)])


def kernel(nodes_or_embeddings, current_node):
    cn = current_node
    if cn.ndim > 1:
        cn = jnp.squeeze(cn, axis=-1)
    cn = cn.astype(jnp.int32)
    table = nodes_or_embeddings.reshape(B * N, E)
    out = _gather_rows(table, cn)
    return out.reshape(B, 1, E)
